# baseline (speedup 1.0000x reference)
import functools

import jax
import jax.numpy as jnp
from jax import lax
from jax.experimental import pallas as pl
from jax.experimental.pallas import tpu as pltpu
from jax.experimental.pallas import tpu_sc as plsc

EPS = 1e-5


_VBLK = 2048


def _ln_body(dx_ref, proc_ref, visit_ref, g_ref, b_ref, gc_ref, bc_ref,
             dx_out, proc_out, visit_out):
    gc = gc_ref[...]
    bc = bc_ref[...]
    half = dx_ref.shape[0] // 2
    for src, dst in ((dx_ref, dx_out), (proc_ref, proc_out)):
        x = src[...]
        m = jnp.mean(x, axis=0, keepdims=True)
        v = jnp.mean((x - m) ** 2, axis=0, keepdims=True)
        y = (x - m) * lax.rsqrt(v + EPS) * gc + bc
        yb = y.astype(jnp.bfloat16)
        lo = lax.bitcast_convert_type(yb[:half], jnp.uint16).astype(jnp.uint32)
        hi = lax.bitcast_convert_type(yb[half:], jnp.uint16).astype(jnp.uint32)
        pk = lax.bitcast_convert_type(lo | (hi << 16), jnp.int32)
        for k in range(_VBLK // 128):
            dst[:, k, :] = pk[:, k * 128:(k + 1) * 128]
    xv = visit_ref[...]
    mv = jnp.mean(xv, axis=-1, keepdims=True)
    vv = jnp.mean((xv - mv) ** 2, axis=-1, keepdims=True)
    visit_out[...] = (xv - mv) * lax.rsqrt(vv + EPS) * g_ref[...] + b_ref[...]


def _ln_tables_t(dx_t, proc_t, visit_table, ln_gamma, ln_beta):
    d, v1 = dx_t.shape
    v_pad = ((v1 + _VBLK - 1) // _VBLK) * _VBLK
    n_blk = v_pad // _VBLK
    tab_spec = pl.BlockSpec((d, _VBLK), lambda i: (0, i))
    pk_spec = pl.BlockSpec((d // 2, _VBLK // 128, 128), lambda i: (0, i, 0))
    one_spec = pl.BlockSpec((1, d), lambda i: (0, 0))
    col_spec = pl.BlockSpec((d, 1), lambda i: (0, 0))
    return pl.pallas_call(
        _ln_body,
        grid=(n_blk,),
        in_specs=[tab_spec, tab_spec, one_spec, one_spec, one_spec,
                  col_spec, col_spec],
        out_specs=[pk_spec, pk_spec, one_spec],
        out_shape=[
            jax.ShapeDtypeStruct((d // 2, v_pad // 128, 128), jnp.int32),
            jax.ShapeDtypeStruct((d // 2, v_pad // 128, 128), jnp.int32),
            jax.ShapeDtypeStruct((1, d), jnp.float32),
        ],
    )(dx_t, proc_t, visit_table,
      ln_gamma.reshape(1, d), ln_beta.reshape(1, d),
      ln_gamma.reshape(d, 1), ln_beta.reshape(d, 1))


def _gather_body(l_dim, b_dim, d_dim,
                 dxp, dx_idx, procp, proc_idx, dx_out, proc_out,
                 vrow, vidx, vout, si0, si1, so0, so1, ss):
    nc = 2
    wid = lax.axis_index("s") * nc + lax.axis_index("c")
    sems_i = (si0, si1)
    sems_o = (so0, so1)
    half = d_dim // 2
    n8 = b_dim // 128

    for p in range(2):
        tab, idxh, outh = (dxp, dx_idx, dx_out) if p == 0 else \
                          (procp, proc_idx, proc_out)
        d_lo = wid
        d_hi = wid + half
        pltpu.async_copy(tab.at[d_lo], vrow, ss).wait()

        def idx_start(s, l):
            pltpu.async_copy(idxh.at[l], vidx.at[s], sems_i[s])

        def idx_wait(s, l):
            pltpu.make_async_copy(idxh.at[l], vidx.at[s], sems_i[s]).wait()

        def out_start(s, l):
            pltpu.async_copy(vout.at[s, 0], outh.at[l, d_lo], sems_o[s])
            pltpu.async_copy(vout.at[s, 1], outh.at[l, d_hi], sems_o[s])

        def out_wait(s, l):
            pltpu.make_async_copy(vout.at[s, 0], outh.at[l, d_lo],
                                  sems_o[s]).wait()
            pltpu.make_async_copy(vout.at[s, 1], outh.at[l, d_hi],
                                  sems_o[s]).wait()

        def gather(s):
            @pl.loop(0, n8)
            def _g(i):
                ivs = [vidx[s, i, pl.ds(j * 16, 16)] for j in range(8)]
                xs = [plsc.load_gather(vrow, [iv >> 7, iv & 127])
                      for iv in ivs]
                for j in range(8):
                    base = i * 128
                    xb = plsc.bitcast(xs[j], jnp.bfloat16)
                    a, b = plsc.unpack(xb, format=plsc.PackFormat.INTERLEAVED)
                    vout[s, 0, pl.ds(base + j * 16, 16)] = a
                    vout[s, 1, pl.ds(base + j * 16, 16)] = b

        idx_start(0, 0)
        idx_start(1, 1)
        for s in (0, 1):
            idx_wait(s, s)
            gather(s)
            out_start(s, s)
            idx_start(s, s + 2)

        @pl.loop(2, l_dim - 2, step=2)
        def _steady(l):
            for s in (0, 1):
                ll = l + s
                out_wait(s, ll - 2)
                idx_wait(s, ll)
                gather(s)
                out_start(s, ll)
                idx_start(s, ll + 2)

        for s in (0, 1):
            ll = l_dim - 2 + s
            out_wait(s, ll - 2)
            idx_wait(s, ll)
            gather(s)
            out_start(s, ll)
        out_wait(0, l_dim - 2)
        out_wait(1, l_dim - 1)


def _sc_gather(dxp, procp, dx_idx_t, proc_idx_t, d_dim):
    half, v128, _ = dxp.shape
    l_dim, b128, _ = dx_idx_t.shape
    b_dim = b128 * 128
    mesh = plsc.VectorSubcoreMesh(core_axis_name="c", subcore_axis_name="s",
                                  num_cores=2, num_subcores=16)
    run = pl.kernel(
        functools.partial(_gather_body, l_dim, b_dim, d_dim),
        out_type=[
            jax.ShapeDtypeStruct((l_dim, d_dim, b_dim), jnp.float32),
            jax.ShapeDtypeStruct((l_dim, d_dim, b_dim), jnp.float32),
        ],
        mesh=mesh,
        scratch_types=[
            pltpu.VMEM((v128, 128), jnp.int32),
            pltpu.VMEM((2, b128, 128), jnp.int32),
            pltpu.VMEM((2, 2, b_dim), jnp.float32),
            pltpu.SemaphoreType.DMA,
            pltpu.SemaphoreType.DMA,
            pltpu.SemaphoreType.DMA,
            pltpu.SemaphoreType.DMA,
            pltpu.SemaphoreType.DMA,
        ],
        compiler_params=pltpu.CompilerParams(needs_layout_passes=False),
    )
    return run(dxp, dx_idx_t, procp, proc_idx_t)


def kernel(dx_table, proc_table, visit_table, ln_gamma, ln_beta,
           dx_ints, proc_ints):
    b, l = dx_ints.shape
    d = dx_table.shape[1]
    dxp, procp, visit_ln = _ln_tables_t(
        dx_table.T, proc_table.T, visit_table, ln_gamma, ln_beta)
    o_dx, o_proc = _sc_gather(dxp, procp,
                              dx_ints.T.astype(jnp.int32).reshape(l, -1, 128),
                              proc_ints.T.astype(jnp.int32).reshape(l, -1, 128),
                              d)
    dx_emb = jnp.transpose(o_dx, (2, 0, 1))
    proc_emb = jnp.transpose(o_proc, (2, 0, 1))
    visit_emb = jnp.broadcast_to(visit_ln.reshape(1, 1, d), (b, 1, d))
    visit_mask = jnp.ones((b, 1), dtype=jnp.float32)
    return (dx_emb, proc_emb, visit_emb, visit_mask)


# speedup vs baseline: 1.2031x; 1.2031x over previous
import functools

import jax
import jax.numpy as jnp
import numpy as np
from jax import lax
from jax.experimental import pallas as pl
from jax.experimental.pallas import tpu as pltpu
from jax.experimental.pallas import tpu_sc as plsc

EPS = 1e-5


_VBLK = 8192


def _ln_body(dx_ref, proc_ref, visit_ref, g_ref, b_ref, gc_ref, bc_ref,
             pm_ref, dx_out, proc_out, visit_out):
    gc = gc_ref[...]
    bc = bc_ref[...]
    half = dx_ref.shape[0] // 2
    pmat = pm_ref[...]
    for src, dst in ((dx_ref, dx_out), (proc_ref, proc_out)):
        x = src[...]
        xp = jax.lax.dot(pmat, x, precision=jax.lax.Precision.HIGHEST)
        m = jnp.mean(xp, axis=0, keepdims=True)
        v = jnp.mean((xp - m) ** 2, axis=0, keepdims=True)
        y = (xp - m) * lax.rsqrt(v + EPS) * gc + bc
        yb = y.astype(jnp.bfloat16)
        lo = lax.bitcast_convert_type(yb[:half], jnp.uint16).astype(jnp.uint32)
        hi = lax.bitcast_convert_type(yb[half:], jnp.uint16).astype(jnp.uint32)
        dst[...] = lax.bitcast_convert_type(lo | (hi << 16), jnp.int32)
    xv = visit_ref[...]
    mv = jnp.mean(xv, axis=-1, keepdims=True)
    vv = jnp.mean((xv - mv) ** 2, axis=-1, keepdims=True)
    visit_out[...] = (xv - mv) * lax.rsqrt(vv + EPS) * g_ref[...] + b_ref[...]


def _ln_tables_t(dx_t, proc_t, visit_table, ln_gamma, ln_beta):
    d, v1 = dx_t.shape
    n_blk = pl.cdiv(v1, _VBLK)
    perm = np.concatenate([np.arange(0, d, 2), np.arange(1, d, 2)])
    gp = ln_gamma[perm].reshape(d, 1)
    bp = ln_beta[perm].reshape(d, 1)
    tab_spec = pl.BlockSpec((d, _VBLK), lambda i: (0, i))
    pk_spec = pl.BlockSpec((d // 2, _VBLK), lambda i: (0, i))
    one_spec = pl.BlockSpec((1, d), lambda i: (0, 0))
    col_spec = pl.BlockSpec((d, 1), lambda i: (0, 0))
    pmat = jnp.asarray(np.eye(d, dtype=np.float32)[perm])
    sq_spec = pl.BlockSpec((d, d), lambda i: (0, 0))
    return pl.pallas_call(
        _ln_body,
        grid=(n_blk,),
        in_specs=[tab_spec, tab_spec, one_spec, one_spec, one_spec,
                  col_spec, col_spec, sq_spec],
        out_specs=[pk_spec, pk_spec, one_spec],
        out_shape=[
            jax.ShapeDtypeStruct((d // 2, v1), jnp.int32),
            jax.ShapeDtypeStruct((d // 2, v1), jnp.int32),
            jax.ShapeDtypeStruct((1, d), jnp.float32),
        ],
    )(dx_t, proc_t, visit_table,
      ln_gamma.reshape(1, d), ln_beta.reshape(1, d), gp, bp, pmat)


def _gather_body(l_dim, b_dim, d_dim,
                 dxp, dx_idx, procp, proc_idx, dx_out, proc_out,
                 vrow, vidx, vout, si0, si1, so0, so1, ss):
    nc = 2
    wid = lax.axis_index("s") * nc + lax.axis_index("c")
    sems_i = (si0, si1)
    sems_o = (so0, so1)
    n8 = b_dim // 128

    for p in range(2):
        tab, idxh, outh = (dxp, dx_idx, dx_out) if p == 0 else \
                          (procp, proc_idx, proc_out)
        d0 = wid * 2
        pltpu.async_copy(tab.at[wid], vrow, ss).wait()

        def idx_start(s, l):
            pltpu.async_copy(idxh.at[l], vidx.at[s], sems_i[s])

        def idx_wait(s, l):
            pltpu.make_async_copy(idxh.at[l], vidx.at[s], sems_i[s]).wait()

        def out_start(s, l):
            pltpu.async_copy(vout.at[s], outh.at[l, pl.ds(d0, 2)], sems_o[s])

        def out_wait(s, l):
            pltpu.make_async_copy(vout.at[s], outh.at[l, pl.ds(d0, 2)],
                                  sems_o[s]).wait()

        def gather(s):
            @pl.loop(0, n8)
            def _g(i):
                base = i * 128
                ivs = [vidx[s, pl.ds(base + j * 16, 16)] for j in range(8)]
                xs = [plsc.load_gather(vrow, [iv]) for iv in ivs]
                for j in range(8):
                    xb = plsc.bitcast(xs[j], jnp.bfloat16)
                    a, b = plsc.unpack(xb, format=plsc.PackFormat.INTERLEAVED)
                    vout[s, 0, pl.ds(base + j * 16, 16)] = a
                    vout[s, 1, pl.ds(base + j * 16, 16)] = b

        idx_start(0, 0)
        idx_start(1, 1)
        for s in (0, 1):
            idx_wait(s, s)
            gather(s)
            out_start(s, s)
            idx_start(s, s + 2)

        @pl.loop(2, l_dim - 2, step=2)
        def _steady(l):
            for s in (0, 1):
                ll = l + s
                out_wait(s, ll - 2)
                idx_wait(s, ll)
                gather(s)
                out_start(s, ll)
                idx_start(s, ll + 2)

        for s in (0, 1):
            ll = l_dim - 2 + s
            out_wait(s, ll - 2)
            idx_wait(s, ll)
            gather(s)
            out_start(s, ll)
        out_wait(0, l_dim - 2)
        out_wait(1, l_dim - 1)


def _sc_gather(dxp, procp, dx_idx_t, proc_idx_t, d_dim):
    half, v1 = dxp.shape
    l_dim, b_dim = dx_idx_t.shape
    mesh = plsc.VectorSubcoreMesh(core_axis_name="c", subcore_axis_name="s",
                                  num_cores=2, num_subcores=16)
    run = pl.kernel(
        functools.partial(_gather_body, l_dim, b_dim, d_dim),
        out_type=[
            jax.ShapeDtypeStruct((l_dim, d_dim, b_dim), jnp.float32),
            jax.ShapeDtypeStruct((l_dim, d_dim, b_dim), jnp.float32),
        ],
        mesh=mesh,
        scratch_types=[
            pltpu.VMEM((v1,), jnp.int32),
            pltpu.VMEM((2, b_dim), jnp.int32),
            pltpu.VMEM((2, 2, b_dim), jnp.float32),
            pltpu.SemaphoreType.DMA,
            pltpu.SemaphoreType.DMA,
            pltpu.SemaphoreType.DMA,
            pltpu.SemaphoreType.DMA,
            pltpu.SemaphoreType.DMA,
        ],
        compiler_params=pltpu.CompilerParams(needs_layout_passes=False),
    )
    return run(dxp, dx_idx_t, procp, proc_idx_t)


def kernel(dx_table, proc_table, visit_table, ln_gamma, ln_beta,
           dx_ints, proc_ints):
    b, l = dx_ints.shape
    d = dx_table.shape[1]
    dxp, procp, visit_ln = _ln_tables_t(
        dx_table.T, proc_table.T, visit_table, ln_gamma, ln_beta)
    o_dx, o_proc = _sc_gather(dxp, procp,
                              dx_ints.T.astype(jnp.int32),
                              proc_ints.T.astype(jnp.int32), d)
    dx_emb = jnp.transpose(o_dx, (2, 0, 1))
    proc_emb = jnp.transpose(o_proc, (2, 0, 1))
    visit_emb = jnp.broadcast_to(visit_ln.reshape(1, 1, d), (b, 1, d))
    visit_mask = jnp.ones((b, 1), dtype=jnp.float32)
    return (dx_emb, proc_emb, visit_emb, visit_mask)


# speedup vs baseline: 1.5295x; 1.2713x over previous
import functools

import jax
import jax.numpy as jnp
import numpy as np
from jax import lax
from jax.experimental import pallas as pl
from jax.experimental.pallas import tpu as pltpu
from jax.experimental.pallas import tpu_sc as plsc

EPS = 1e-5


_VBLK = 8192


def _ln_body(dx_ref, proc_ref, visit_ref, g_ref, b_ref, gc_ref, bc_ref,
             pm_ref, dx_out, proc_out, visit_out):
    gc = gc_ref[...]
    bc = bc_ref[...]
    half = dx_ref.shape[0] // 2
    pmat = pm_ref[...]
    for src, dst in ((dx_ref, dx_out), (proc_ref, proc_out)):
        x = src[...]
        xp = jax.lax.dot(pmat, x, precision=jax.lax.Precision.HIGHEST)
        m = jnp.mean(xp, axis=0, keepdims=True)
        v = jnp.mean((xp - m) ** 2, axis=0, keepdims=True)
        y = (xp - m) * lax.rsqrt(v + EPS) * gc + bc
        yb = y.astype(jnp.bfloat16)
        lo = lax.bitcast_convert_type(yb[:half], jnp.uint16).astype(jnp.uint32)
        hi = lax.bitcast_convert_type(yb[half:], jnp.uint16).astype(jnp.uint32)
        dst[...] = lax.bitcast_convert_type(lo | (hi << 16), jnp.int32)
    xv = visit_ref[...]
    mv = jnp.mean(xv, axis=-1, keepdims=True)
    vv = jnp.mean((xv - mv) ** 2, axis=-1, keepdims=True)
    visit_out[...] = (xv - mv) * lax.rsqrt(vv + EPS) * g_ref[...] + b_ref[...]


def _ln_tables_t(dx_t, proc_t, visit_table, ln_gamma, ln_beta):
    d, v1 = dx_t.shape
    n_blk = pl.cdiv(v1, _VBLK)
    perm = np.concatenate([np.arange(0, d, 2), np.arange(1, d, 2)])
    gp = ln_gamma[perm].reshape(d, 1)
    bp = ln_beta[perm].reshape(d, 1)
    tab_spec = pl.BlockSpec((d, _VBLK), lambda i: (0, i))
    pk_spec = pl.BlockSpec((d // 2, _VBLK), lambda i: (0, i))
    one_spec = pl.BlockSpec((1, d), lambda i: (0, 0))
    col_spec = pl.BlockSpec((d, 1), lambda i: (0, 0))
    pmat = jnp.asarray(np.eye(d, dtype=np.float32)[perm])
    sq_spec = pl.BlockSpec((d, d), lambda i: (0, 0))
    return pl.pallas_call(
        _ln_body,
        grid=(n_blk,),
        in_specs=[tab_spec, tab_spec, one_spec, one_spec, one_spec,
                  col_spec, col_spec, sq_spec],
        out_specs=[pk_spec, pk_spec, one_spec],
        out_shape=[
            jax.ShapeDtypeStruct((d // 2, v1), jnp.int32),
            jax.ShapeDtypeStruct((d // 2, v1), jnp.int32),
            jax.ShapeDtypeStruct((1, d), jnp.float32),
        ],
    )(dx_t, proc_t, visit_table,
      ln_gamma.reshape(1, d), ln_beta.reshape(1, d), gp, bp, pmat)


_LBLK = 8


def _gather_body(l_dim, b_dim, d_dim,
                 dxp, dx_idx, procp, proc_idx, dx_out, proc_out,
                 vrow, vidx, vout, sp_idx, si0, si1, so0, so1, ss, sb0, sb1):
    nc = 2
    sid = lax.axis_index("s")
    wid = sid * nc + lax.axis_index("c")
    sems_i = (si0, si1)
    sems_o = (so0, so1)
    sems_b = (sb0, sb1)
    n8 = b_dim // 128
    n_blk = l_dim // _LBLK
    d0 = wid * 2

    def idx_start(s, ss_, j):
        pltpu.async_copy(sp_idx.at[ss_, j], vidx.at[s], sems_i[s])

    def idx_wait(s, ss_, j):
        pltpu.make_async_copy(sp_idx.at[ss_, j], vidx.at[s],
                              sems_i[s]).wait()

    def out_start(outh, s, l):
        pltpu.async_copy(vout.at[s], outh.at[l, pl.ds(d0, 2)], sems_o[s])

    def out_drain(s):
        pltpu.make_async_copy(vout.at[s], dx_out.at[0, pl.ds(d0, 2)],
                              sems_o[s]).wait()

    def gather(s):
        @pl.loop(0, n8)
        def _g(i):
            base = i * 128
            ivs = [vidx[s, pl.ds(base + j * 16, 16)] for j in range(8)]
            xs = [plsc.load_gather(vrow, [iv]) for iv in ivs]
            for j in range(8):
                xb = plsc.bitcast(xs[j], jnp.bfloat16)
                a, b = plsc.unpack(xb, format=plsc.PackFormat.INTERLEAVED)
                vout[s, 0, pl.ds(base + j * 16, 16)] = a
                vout[s, 1, pl.ds(base + j * 16, 16)] = b

    def emit_rows(ss_, l0, outh, skip_first_drains):
        idx_start(0, ss_, 0)
        idx_start(1, ss_, 1)

        @pl.loop(0, _LBLK, step=2)
        def _rows(j):
            for s in (0, 1):
                jj = j + s
                if skip_first_drains:
                    @pl.when(jj >= 2)
                    def _d(s=s):
                        out_drain(s)
                else:
                    out_drain(s)
                idx_wait(s, ss_, jj)
                gather(s)
                out_start(outh, s, l0 + jj)

                @pl.when(jj + 2 < _LBLK)
                def _pf(s=s, jj=jj):
                    idx_start(s, ss_, jj + 2)

    def emit_block(idxh, outh, blk, ss_, skip_first_drains):
        def src(b):
            return idxh.at[pl.ds(b * _LBLK, _LBLK)]

        @pl.when(sid == 0)
        def _w():
            pltpu.make_async_copy(src(blk), sp_idx.at[ss_],
                                  sems_b[ss_]).wait()
        plsc.subcore_barrier()

        emit_rows(ss_, blk * _LBLK, outh, skip_first_drains)

        plsc.subcore_barrier()

        if isinstance(blk, int) and blk + 2 >= n_blk:
            return

        @pl.when((sid == 0) & (blk + 2 < n_blk))
        def _n():
            pltpu.async_copy(src(blk + 2), sp_idx.at[ss_], sems_b[ss_])

    @pl.when(sid == 0)
    def _prime():
        pltpu.async_copy(dx_idx.at[pl.ds(0, _LBLK)], sp_idx.at[0], sb0)
        pltpu.async_copy(dx_idx.at[pl.ds(_LBLK, _LBLK)], sp_idx.at[1], sb1)

    for p in range(2):
        tab, idxh, outh = (dxp, dx_idx, dx_out) if p == 0 else \
                          (procp, proc_idx, proc_out)
        pltpu.async_copy(tab.at[wid], vrow, ss).wait()

        emit_block(idxh, outh, 0, 0, p == 0)

        @pl.loop(1, n_blk, step=2)
        def _blocks(blk):
            emit_block(idxh, outh, blk, 1, False)
            emit_block(idxh, outh, blk + 1, 0, False)

        if p == 0:
            @pl.when(sid == 0)
            def _nextpass():
                pltpu.async_copy(proc_idx.at[pl.ds(0, _LBLK)],
                                 sp_idx.at[0], sb0)
                pltpu.async_copy(proc_idx.at[pl.ds(_LBLK, _LBLK)],
                                 sp_idx.at[1], sb1)

    out_drain(0)
    out_drain(1)


def _sc_gather(dxp, procp, dx_idx_t, proc_idx_t, d_dim):
    half, v1 = dxp.shape
    l_dim, b_dim = dx_idx_t.shape
    mesh = plsc.VectorSubcoreMesh(core_axis_name="c", subcore_axis_name="s",
                                  num_cores=2, num_subcores=16)
    run = pl.kernel(
        functools.partial(_gather_body, l_dim, b_dim, d_dim),
        out_type=[
            jax.ShapeDtypeStruct((l_dim, d_dim, b_dim), jnp.float32),
            jax.ShapeDtypeStruct((l_dim, d_dim, b_dim), jnp.float32),
        ],
        mesh=mesh,
        scratch_types=[
            pltpu.VMEM((v1,), jnp.int32),
            pltpu.VMEM((2, b_dim), jnp.int32),
            pltpu.VMEM((2, 2, b_dim), jnp.float32),
            pltpu.VMEM_SHARED((2, _LBLK, b_dim), jnp.int32),
            pltpu.SemaphoreType.DMA,
            pltpu.SemaphoreType.DMA,
            pltpu.SemaphoreType.DMA,
            pltpu.SemaphoreType.DMA,
            pltpu.SemaphoreType.DMA,
            pltpu.SemaphoreType.DMA,
            pltpu.SemaphoreType.DMA,
        ],
        compiler_params=pltpu.CompilerParams(needs_layout_passes=False),
    )
    return run(dxp, dx_idx_t, procp, proc_idx_t)


def kernel(dx_table, proc_table, visit_table, ln_gamma, ln_beta,
           dx_ints, proc_ints):
    b, l = dx_ints.shape
    d = dx_table.shape[1]
    dxp, procp, visit_ln = _ln_tables_t(
        dx_table.T, proc_table.T, visit_table, ln_gamma, ln_beta)
    o_dx, o_proc = _sc_gather(dxp, procp,
                              dx_ints.T.astype(jnp.int32),
                              proc_ints.T.astype(jnp.int32), d)
    dx_emb = jnp.transpose(o_dx, (2, 0, 1))
    proc_emb = jnp.transpose(o_proc, (2, 0, 1))
    visit_emb = jnp.broadcast_to(visit_ln.reshape(1, 1, d), (b, 1, d))
    visit_mask = jnp.ones((b, 1), dtype=jnp.float32)
    return (dx_emb, proc_emb, visit_emb, visit_mask)


# speedup vs baseline: 1.5942x; 1.0423x over previous
import functools

import jax
import jax.numpy as jnp
import numpy as np
from jax import lax
from jax.experimental import pallas as pl
from jax.experimental.pallas import tpu as pltpu
from jax.experimental.pallas import tpu_sc as plsc

EPS = 1e-5


_VBLK = 8192


def _ln_pack(x, gc, bc, pmat):
    half = x.shape[0] // 2
    xp = jax.lax.dot(pmat, x, precision=jax.lax.Precision.HIGHEST)
    m = jnp.mean(xp, axis=0, keepdims=True)
    v = jnp.mean((xp - m) ** 2, axis=0, keepdims=True)
    y = (xp - m) * lax.rsqrt(v + EPS) * gc + bc
    yb = y.astype(jnp.bfloat16)
    lo = lax.bitcast_convert_type(yb[:half], jnp.uint16).astype(jnp.uint32)
    hi = lax.bitcast_convert_type(yb[half:], jnp.uint16).astype(jnp.uint32)
    return lax.bitcast_convert_type(lo | (hi << 16), jnp.int32)


def _ln_body_visit(tab_ref, visit_ref, g_ref, b_ref, gc_ref, bc_ref, pm_ref,
                   pk_out, visit_out):
    pk_out[...] = _ln_pack(tab_ref[...], gc_ref[...], bc_ref[...], pm_ref[...])
    xv = visit_ref[...]
    mv = jnp.mean(xv, axis=-1, keepdims=True)
    vv = jnp.mean((xv - mv) ** 2, axis=-1, keepdims=True)
    visit_out[...] = (xv - mv) * lax.rsqrt(vv + EPS) * g_ref[...] + b_ref[...]


def _ln_body(tab_ref, gc_ref, bc_ref, pm_ref, pk_out):
    pk_out[...] = _ln_pack(tab_ref[...], gc_ref[...], bc_ref[...], pm_ref[...])


def _ln_table_t(tab_t, ln_gamma, ln_beta, visit_table=None):
    d, v1 = tab_t.shape
    n_blk = pl.cdiv(v1, _VBLK)
    perm = np.concatenate([np.arange(0, d, 2), np.arange(1, d, 2)])
    gp = ln_gamma[perm].reshape(d, 1)
    bp = ln_beta[perm].reshape(d, 1)
    pmat = jnp.asarray(np.eye(d, dtype=np.float32)[perm])
    tab_spec = pl.BlockSpec((d, _VBLK), lambda i: (0, i))
    pk_spec = pl.BlockSpec((d // 2, _VBLK), lambda i: (0, i))
    one_spec = pl.BlockSpec((1, d), lambda i: (0, 0))
    col_spec = pl.BlockSpec((d, 1), lambda i: (0, 0))
    sq_spec = pl.BlockSpec((d, d), lambda i: (0, 0))
    pk_shape = jax.ShapeDtypeStruct((d // 2, v1), jnp.int32)
    if visit_table is None:
        return pl.pallas_call(
            _ln_body,
            grid=(n_blk,),
            in_specs=[tab_spec, col_spec, col_spec, sq_spec],
            out_specs=[pk_spec],
            out_shape=[pk_shape],
        )(tab_t, gp, bp, pmat)[0]
    return pl.pallas_call(
        _ln_body_visit,
        grid=(n_blk,),
        in_specs=[tab_spec, one_spec, one_spec, one_spec,
                  col_spec, col_spec, sq_spec],
        out_specs=[pk_spec, one_spec],
        out_shape=[pk_shape, jax.ShapeDtypeStruct((1, d), jnp.float32)],
    )(tab_t, visit_table, ln_gamma.reshape(1, d), ln_beta.reshape(1, d),
      gp, bp, pmat)


_LBLK = 8


def _gather_body(l_dim, b_dim,
                 tabp, idxh, outh,
                 vrow, vidx, vout, sp_idx, si0, si1, so0, so1, ss, sb0, sb1):
    nc = 2
    sid = lax.axis_index("s")
    wid = sid * nc + lax.axis_index("c")
    sems_i = (si0, si1)
    sems_o = (so0, so1)
    sems_b = (sb0, sb1)
    n8 = b_dim // 128
    n_blk = l_dim // _LBLK
    d0 = wid * 2

    def idx_start(s, ss_, j):
        pltpu.async_copy(sp_idx.at[ss_, j], vidx.at[s], sems_i[s])

    def idx_wait(s, ss_, j):
        pltpu.make_async_copy(sp_idx.at[ss_, j], vidx.at[s],
                              sems_i[s]).wait()

    def out_start(s, l):
        pltpu.async_copy(vout.at[s], outh.at[l, pl.ds(d0, 2)], sems_o[s])

    def out_drain(s):
        pltpu.make_async_copy(vout.at[s], outh.at[0, pl.ds(d0, 2)],
                              sems_o[s]).wait()

    def gather(s):
        @pl.loop(0, n8)
        def _g(i):
            base = i * 128
            ivs = [vidx[s, pl.ds(base + j * 16, 16)] for j in range(8)]
            xs = [plsc.load_gather(vrow, [iv]) for iv in ivs]
            for j in range(8):
                xb = plsc.bitcast(xs[j], jnp.bfloat16)
                a, b = plsc.unpack(xb, format=plsc.PackFormat.INTERLEAVED)
                vout[s, 0, pl.ds(base + j * 16, 16)] = a
                vout[s, 1, pl.ds(base + j * 16, 16)] = b

    def emit_rows(ss_, l0, skip_first_drains):
        idx_start(0, ss_, 0)
        idx_start(1, ss_, 1)

        @pl.loop(0, _LBLK, step=2)
        def _rows(j):
            for s in (0, 1):
                jj = j + s
                if skip_first_drains:
                    @pl.when(jj >= 2)
                    def _d(s=s):
                        out_drain(s)
                else:
                    out_drain(s)
                idx_wait(s, ss_, jj)
                gather(s)
                out_start(s, l0 + jj)

                @pl.when(jj + 2 < _LBLK)
                def _pf(s=s, jj=jj):
                    idx_start(s, ss_, jj + 2)

    def emit_block(blk, ss_, skip_first_drains):
        def src(b):
            return idxh.at[pl.ds(b * _LBLK, _LBLK)]

        @pl.when(sid == 0)
        def _w():
            pltpu.make_async_copy(src(blk), sp_idx.at[ss_],
                                  sems_b[ss_]).wait()
        plsc.subcore_barrier()

        emit_rows(ss_, blk * _LBLK, skip_first_drains)

        plsc.subcore_barrier()

        if isinstance(blk, int) and blk + 2 >= n_blk:
            return

        @pl.when((sid == 0) & (blk + 2 < n_blk))
        def _n():
            pltpu.async_copy(src(blk + 2), sp_idx.at[ss_], sems_b[ss_])

    @pl.when(sid == 0)
    def _prime():
        pltpu.async_copy(idxh.at[pl.ds(0, _LBLK)], sp_idx.at[0], sb0)
        pltpu.async_copy(idxh.at[pl.ds(_LBLK, _LBLK)], sp_idx.at[1], sb1)

    pltpu.async_copy(tabp.at[wid], vrow, ss).wait()

    emit_block(0, 0, True)

    @pl.loop(1, n_blk, step=2)
    def _blocks(blk):
        emit_block(blk, 1, False)
        emit_block(blk + 1, 0, False)

    out_drain(0)
    out_drain(1)


def _sc_gather(tabp, idx_t, d_dim):
    half, v1 = tabp.shape
    l_dim, b_dim = idx_t.shape
    mesh = plsc.VectorSubcoreMesh(core_axis_name="c", subcore_axis_name="s",
                                  num_cores=2, num_subcores=16)
    run = pl.kernel(
        functools.partial(_gather_body, l_dim, b_dim),
        out_type=[
            jax.ShapeDtypeStruct((l_dim, d_dim, b_dim), jnp.float32),
        ],
        mesh=mesh,
        scratch_types=[
            pltpu.VMEM((v1,), jnp.int32),
            pltpu.VMEM((2, b_dim), jnp.int32),
            pltpu.VMEM((2, 2, b_dim), jnp.float32),
            pltpu.VMEM_SHARED((2, _LBLK, b_dim), jnp.int32),
            pltpu.SemaphoreType.DMA,
            pltpu.SemaphoreType.DMA,
            pltpu.SemaphoreType.DMA,
            pltpu.SemaphoreType.DMA,
            pltpu.SemaphoreType.DMA,
            pltpu.SemaphoreType.DMA,
            pltpu.SemaphoreType.DMA,
        ],
        compiler_params=pltpu.CompilerParams(needs_layout_passes=False),
    )
    return run(tabp, idx_t)[0]


def kernel(dx_table, proc_table, visit_table, ln_gamma, ln_beta,
           dx_ints, proc_ints):
    b, l = dx_ints.shape
    d = dx_table.shape[1]
    dxp, visit_ln = _ln_table_t(dx_table.T, ln_gamma, ln_beta, visit_table)
    o_dx = _sc_gather(dxp, dx_ints.T.astype(jnp.int32), d)
    procp = _ln_table_t(proc_table.T, ln_gamma, ln_beta)
    o_proc = _sc_gather(procp, proc_ints.T.astype(jnp.int32), d)
    dx_emb = jnp.transpose(o_dx, (2, 0, 1))
    proc_emb = jnp.transpose(o_proc, (2, 0, 1))
    visit_emb = jnp.broadcast_to(visit_ln.reshape(1, 1, d), (b, 1, d))
    visit_mask = jnp.ones((b, 1), dtype=jnp.float32)
    return (dx_emb, proc_emb, visit_emb, visit_mask)
